# baseline (speedup 1.0000x reference)
import functools

import jax
import jax.numpy as jnp
from jax import lax
from jax.experimental import pallas as pl
from jax.experimental.pallas import tpu as pltpu
from jax.experimental.pallas import tpu_sc as plsc


def _sc_relpos_copy(pe, n, off):
    V, D = pe.shape
    info = plsc.get_sparse_core_info()
    NC, NS = info.num_cores, info.num_subcores
    NW = NC * NS
    assert n % NW == 0
    rows_per_w = n // NW

    pe_flat = pe.reshape(V * D)

    mesh = plsc.VectorSubcoreMesh(core_axis_name="c", subcore_axis_name="s")

    @functools.partial(
        pl.kernel,
        out_type=jax.ShapeDtypeStruct((n, n * D), jnp.float32),
        mesh=mesh,
    )
    def k(pe_hbm, out_hbm):
        wid = lax.axis_index("s") * NC + lax.axis_index("c")

        def body(r, carry):
            i = wid * rows_per_w + r
            pltpu.sync_copy(
                pe_hbm.at[pl.ds((off - i) * D, n * D)], out_hbm.at[i]
            )
            return carry

        lax.fori_loop(0, rows_per_w, body, 0)

    return k(pe_flat).reshape(n, n, D)


def kernel(x, q_len, pe):
    n = x.shape[1]
    V = pe.shape[0]
    off = (V + 1) // 2 - 1
    return _sc_relpos_copy(pe, n, off)


# speedup vs baseline: 20.1775x; 20.1775x over previous
import functools

import jax
import jax.numpy as jnp
from jax import lax
from jax.experimental import pallas as pl
from jax.experimental.pallas import tpu as pltpu
from jax.experimental.pallas import tpu_sc as plsc


def _sc_relpos_copy(pe, n, off):
    V, D = pe.shape
    info = plsc.get_sparse_core_info()
    NC, NS = info.num_cores, info.num_subcores
    NW = NC * NS
    assert n % NW == 0
    rows_per_w = n // NW

    lo = ((off - n + 1) // 8) * 8
    stage_rows = 2 * n
    stripe = stage_rows // NS

    mesh = plsc.VectorSubcoreMesh(core_axis_name="c", subcore_axis_name="s")

    @functools.partial(
        pl.kernel,
        out_type=jax.ShapeDtypeStruct((n, n, D), jnp.float32),
        mesh=mesh,
        scratch_types=[
            pltpu.VMEM_SHARED((stage_rows, D), jnp.float32),
            pltpu.SemaphoreType.DMA,
        ],
        compiler_params=pltpu.CompilerParams(use_tc_tiling_on_sc=False),
    )
    def k(pe_hbm, out_hbm, shared, sem):
        cid = lax.axis_index("c")
        sid = lax.axis_index("s")
        wid = sid * NC + cid

        pltpu.sync_copy(
            pe_hbm.at[pl.ds(lo + sid * stripe, stripe)],
            shared.at[pl.ds(sid * stripe, stripe)],
        )
        plsc.subcore_barrier()

        i0 = wid * rows_per_w
        copies = []
        for r in range(rows_per_w):
            i = i0 + r
            copies.append(
                pltpu.async_copy(
                    shared.at[pl.ds(off - i - lo, n)], out_hbm.at[i], sem
                )
            )
        for c in copies:
            c.wait()

    return k(pe)


def kernel(x, q_len, pe):
    n = x.shape[1]
    V = pe.shape[0]
    off = (V + 1) // 2 - 1
    return _sc_relpos_copy(pe, n, off)
